# baseline (speedup 1.0000x reference)
import jax
import jax.numpy as jnp
import numpy as np
from jax.experimental import pallas as pl
from jax.experimental.pallas import tpu as pltpu

_ROWS = 64
_W = 512
_K = 50
_TINY = np.float32(np.finfo(np.float32).tiny)
def _threefry_bits(a):
    ks1 = jnp.uint32(42)
    ks2 = jnp.uint32(42 ^ 0x1BD11BDA)

    def rotl(x, d):
        return (x << jnp.uint32(d)) | (x >> jnp.uint32(32 - d))

    def four_rounds(x0, x1, rots):
        for r in rots:
            x0 = x0 + x1
            x1 = rotl(x1, r)
            x1 = x0 ^ x1
        return x0, x1

    r1 = (13, 15, 26, 6)
    r2 = (17, 29, 16, 24)
    x0 = a
    x1 = rotl(a, 13) ^ a
    x0, x1 = four_rounds(x0, x1, (15, 26, 6))
    x0 = x0 + ks1
    x1 = x1 + (ks2 + jnp.uint32(1))
    x0, x1 = four_rounds(x0, x1, r2)
    x0 = x0 + ks2
    x1 = x1 + jnp.uint32(2)
    x0, x1 = four_rounds(x0, x1, r1)
    x0 = x0
    x1 = x1 + (ks1 + jnp.uint32(3))
    x0, x1 = four_rounds(x0, x1, r2)
    x0 = x0 + ks1
    x1 = x1 + (ks2 + jnp.uint32(4))
    x0, x1 = four_rounds(x0, x1, r1)
    x0 = x0 + ks2
    x1 = x1 + jnp.uint32(5)
    return x0 ^ x1


def _make_kernel(vocab, n_steps):
    w = _W
    shift = int(np.log2(w))

    def body(x_ref, o_ref, best_ref, btid_ref, jb_ref, bits_ref, w_ref,
             xprev_ref):
        i = pl.program_id(0)
        s = pl.program_id(1)

        @pl.when(s == 0)
        def _init():
            sub = jax.lax.broadcasted_iota(jnp.int32, (_ROWS, w), 0)
            lane = jax.lax.broadcasted_iota(jnp.int32, (_ROWS, w), 1)
            row = i * _ROWS + sub
            jb_ref[...] = (row * vocab + lane + 42).astype(jnp.uint32)
            best_ref[...] = jnp.full((_ROWS, w), -jnp.inf, jnp.float32)
            btid_ref[...] = jnp.zeros((_ROWS, w), jnp.int32)

        lane = jax.lax.broadcasted_iota(jnp.int32, (_ROWS, w), 1)
        base = s * _K

        for c in range(_K):
            qf = base + c - 2

            wv = w_ref[...]
            g = -jnp.log(wv)
            if c == 0:
                xs = xprev_ref[:, 0:w]
            elif c == 1:
                xs = xprev_ref[:, w:2 * w]
            else:
                xs = x_ref[:, (c - 2) * w:(c - 1) * w]
            z = g + xs
            bound = jnp.where(
                jnp.logical_and(qf >= 0, qf * w < vocab),
                vocab - qf * w, 0)
            z = jnp.where(lane < bound, z, -jnp.inf)
            prev = best_ref[...]
            b = jnp.maximum(prev, z)
            m = b != prev
            best_ref[...] = b
            btid_ref[...] = jnp.where(m, qf, btid_ref[...])

            bits = bits_ref[...]
            fbits = (bits >> jnp.uint32(9)) | jnp.uint32(0x3F800000)
            f = jax.lax.bitcast_convert_type(fbits, jnp.float32) \
                - jnp.float32(1.0)
            u = jnp.maximum(f, _TINY)
            w_ref[...] = -jnp.log(u)

            a = jb_ref[...] + ((base + c) * w).astype(jnp.uint32)
            bits_ref[...] = _threefry_bits(a)

        xprev_ref[...] = x_ref[:, (_K - 2) * w:_K * w]

        @pl.when(s == n_steps - 1)
        def _fin():
            bb = best_ref[...]
            col = (btid_ref[...] << shift) + lane
            gmax = jnp.max(bb, axis=1, keepdims=True)
            tok = jnp.min(jnp.where(bb == gmax, col, vocab), axis=1,
                          keepdims=True)
            o_ref[...] = tok

    return body


def kernel(logits):
    b, l, vocab = logits.shape
    rows = b * l
    x = logits.reshape(rows, vocab)
    n_chunks = pl.cdiv(vocab, _W)
    n_steps = pl.cdiv(n_chunks + 2, _K)
    xblocks = pl.cdiv(vocab, _K * _W)

    def x_map(i, s):
        return (i, jnp.minimum(s, xblocks - 1))

    out = pl.pallas_call(
        _make_kernel(vocab, n_steps),
        grid=(rows // _ROWS, n_steps),
        in_specs=[pl.BlockSpec((_ROWS, _K * _W), x_map)],
        out_specs=pl.BlockSpec((_ROWS, 1), lambda i, s: (i, 0)),
        out_shape=jax.ShapeDtypeStruct((rows, 1), jnp.int32),
        scratch_shapes=[
            pltpu.VMEM((_ROWS, _W), jnp.float32),
            pltpu.VMEM((_ROWS, _W), jnp.int32),
            pltpu.VMEM((_ROWS, _W), jnp.uint32),
            pltpu.VMEM((_ROWS, _W), jnp.uint32),
            pltpu.VMEM((_ROWS, _W), jnp.float32),
            pltpu.VMEM((_ROWS, 2 * _W), jnp.float32),
        ],
    )(x)
    return out.reshape(b, l)


# speedup vs baseline: 1.0142x; 1.0142x over previous
import jax
import jax.numpy as jnp
import numpy as np
from jax.experimental import pallas as pl
from jax.experimental.pallas import tpu as pltpu

_ROWS = 32
_W = 512
_K = 66
_TINY = np.float32(np.finfo(np.float32).tiny)
def _threefry_bits(a):
    ks1 = jnp.uint32(42)
    ks2 = jnp.uint32(42 ^ 0x1BD11BDA)

    def rotl(x, d):
        return (x << jnp.uint32(d)) | (x >> jnp.uint32(32 - d))

    def four_rounds(x0, x1, rots):
        for r in rots:
            x0 = x0 + x1
            x1 = rotl(x1, r)
            x1 = x0 ^ x1
        return x0, x1

    r1 = (13, 15, 26, 6)
    r2 = (17, 29, 16, 24)
    x0 = a
    x1 = rotl(a, 13) ^ a
    x0, x1 = four_rounds(x0, x1, (15, 26, 6))
    x0 = x0 + ks1
    x1 = x1 + (ks2 + jnp.uint32(1))
    x0, x1 = four_rounds(x0, x1, r2)
    x0 = x0 + ks2
    x1 = x1 + jnp.uint32(2)
    x0, x1 = four_rounds(x0, x1, r1)
    x0 = x0
    x1 = x1 + (ks1 + jnp.uint32(3))
    x0, x1 = four_rounds(x0, x1, r2)
    x0 = x0 + ks1
    x1 = x1 + (ks2 + jnp.uint32(4))
    x0, x1 = four_rounds(x0, x1, r1)
    x0 = x0 + ks2
    x1 = x1 + jnp.uint32(5)
    return x0 ^ x1


def _make_kernel(vocab, n_steps):
    w = _W
    shift = int(np.log2(w))

    def body(x_ref, o_ref, best_ref, btid_ref, jb_ref, bits_ref, w_ref,
             xprev_ref):
        i = pl.program_id(0)
        s = pl.program_id(1)

        @pl.when(s == 0)
        def _init():
            sub = jax.lax.broadcasted_iota(jnp.int32, (_ROWS, w), 0)
            lane = jax.lax.broadcasted_iota(jnp.int32, (_ROWS, w), 1)
            row = i * _ROWS + sub
            jb_ref[...] = (row * vocab + lane + 42).astype(jnp.uint32)
            best_ref[...] = jnp.full((_ROWS, w), -jnp.inf, jnp.float32)
            btid_ref[...] = jnp.zeros((_ROWS, w), jnp.int32)

        lane = jax.lax.broadcasted_iota(jnp.int32, (_ROWS, w), 1)
        base = s * _K

        for c in range(_K):
            qf = base + c - 2

            wv = w_ref[...]
            g = -jnp.log(wv)
            if c == 0:
                xs = xprev_ref[:, 0:w]
            elif c == 1:
                xs = xprev_ref[:, w:2 * w]
            else:
                xs = x_ref[:, (c - 2) * w:(c - 1) * w]
            z = g + xs
            bound = jnp.where(
                jnp.logical_and(qf >= 0, qf * w < vocab),
                vocab - qf * w, 0)
            z = jnp.where(lane < bound, z, -jnp.inf)
            prev = best_ref[...]
            b = jnp.maximum(prev, z)
            m = b != prev
            best_ref[...] = b
            btid_ref[...] = jnp.where(m, qf, btid_ref[...])

            bits = bits_ref[...]
            fbits = (bits >> jnp.uint32(9)) | jnp.uint32(0x3F800000)
            f = jax.lax.bitcast_convert_type(fbits, jnp.float32) \
                - jnp.float32(1.0)
            u = jnp.maximum(f, _TINY)
            w_ref[...] = -jnp.log(u)

            a = jb_ref[...] + ((base + c) * w).astype(jnp.uint32)
            bits_ref[...] = _threefry_bits(a)

        xprev_ref[...] = x_ref[:, (_K - 2) * w:_K * w]

        @pl.when(s == n_steps - 1)
        def _fin():
            bb = best_ref[...]
            col = (btid_ref[...] << shift) + lane
            gmax = jnp.max(bb, axis=1, keepdims=True)
            tok = jnp.min(jnp.where(bb == gmax, col, vocab), axis=1,
                          keepdims=True)
            o_ref[...] = tok

    return body


def kernel(logits):
    b, l, vocab = logits.shape
    rows = b * l
    x = logits.reshape(rows, vocab)
    n_chunks = pl.cdiv(vocab, _W)
    n_steps = pl.cdiv(n_chunks + 2, _K)
    xblocks = pl.cdiv(vocab, _K * _W)

    def x_map(i, s):
        return (i, jnp.minimum(s, xblocks - 1))

    out = pl.pallas_call(
        _make_kernel(vocab, n_steps),
        grid=(rows // _ROWS, n_steps),
        in_specs=[pl.BlockSpec((_ROWS, _K * _W), x_map)],
        out_specs=pl.BlockSpec((_ROWS, 1), lambda i, s: (i, 0)),
        out_shape=jax.ShapeDtypeStruct((rows, 1), jnp.int32),
        scratch_shapes=[
            pltpu.VMEM((_ROWS, _W), jnp.float32),
            pltpu.VMEM((_ROWS, _W), jnp.int32),
            pltpu.VMEM((_ROWS, _W), jnp.uint32),
            pltpu.VMEM((_ROWS, _W), jnp.uint32),
            pltpu.VMEM((_ROWS, _W), jnp.float32),
            pltpu.VMEM((_ROWS, 2 * _W), jnp.float32),
        ],
    )(x)
    return out.reshape(b, l)


# speedup vs baseline: 1.0156x; 1.0014x over previous
import jax
import jax.numpy as jnp
import numpy as np
from jax.experimental import pallas as pl
from jax.experimental.pallas import tpu as pltpu

_ROWS = 32
_W = 512
_K = 99
_TINY = np.float32(np.finfo(np.float32).tiny)
def _threefry_bits(a):
    ks1 = jnp.uint32(42)
    ks2 = jnp.uint32(42 ^ 0x1BD11BDA)

    def rotl(x, d):
        return (x << jnp.uint32(d)) | (x >> jnp.uint32(32 - d))

    def four_rounds(x0, x1, rots):
        for r in rots:
            x0 = x0 + x1
            x1 = rotl(x1, r)
            x1 = x0 ^ x1
        return x0, x1

    r1 = (13, 15, 26, 6)
    r2 = (17, 29, 16, 24)
    x0 = a
    x1 = rotl(a, 13) ^ a
    x0, x1 = four_rounds(x0, x1, (15, 26, 6))
    x0 = x0 + ks1
    x1 = x1 + (ks2 + jnp.uint32(1))
    x0, x1 = four_rounds(x0, x1, r2)
    x0 = x0 + ks2
    x1 = x1 + jnp.uint32(2)
    x0, x1 = four_rounds(x0, x1, r1)
    x0 = x0
    x1 = x1 + (ks1 + jnp.uint32(3))
    x0, x1 = four_rounds(x0, x1, r2)
    x0 = x0 + ks1
    x1 = x1 + (ks2 + jnp.uint32(4))
    x0, x1 = four_rounds(x0, x1, r1)
    x0 = x0 + ks2
    x1 = x1 + jnp.uint32(5)
    return x0 ^ x1


def _make_kernel(vocab, n_steps):
    w = _W
    shift = int(np.log2(w))

    def body(x_ref, o_ref, best_ref, btid_ref, jb_ref, bits_ref, w_ref,
             xprev_ref):
        i = pl.program_id(0)
        s = pl.program_id(1)

        @pl.when(s == 0)
        def _init():
            sub = jax.lax.broadcasted_iota(jnp.int32, (_ROWS, w), 0)
            lane = jax.lax.broadcasted_iota(jnp.int32, (_ROWS, w), 1)
            row = i * _ROWS + sub
            jb_ref[...] = (row * vocab + lane + 42).astype(jnp.uint32)
            best_ref[...] = jnp.full((_ROWS, w), -jnp.inf, jnp.float32)
            btid_ref[...] = jnp.zeros((_ROWS, w), jnp.int32)

        lane = jax.lax.broadcasted_iota(jnp.int32, (_ROWS, w), 1)
        base = s * _K

        for c in range(_K):
            qf = base + c - 2

            wv = w_ref[...]
            g = -jnp.log(wv)
            if c == 0:
                xs = xprev_ref[:, 0:w]
            elif c == 1:
                xs = xprev_ref[:, w:2 * w]
            else:
                xs = x_ref[:, (c - 2) * w:(c - 1) * w]
            z = g + xs
            bound = jnp.where(
                jnp.logical_and(qf >= 0, qf * w < vocab),
                vocab - qf * w, 0)
            z = jnp.where(lane < bound, z, -jnp.inf)
            prev = best_ref[...]
            b = jnp.maximum(prev, z)
            m = b != prev
            best_ref[...] = b
            btid_ref[...] = jnp.where(m, qf, btid_ref[...])

            bits = bits_ref[...]
            fbits = (bits >> jnp.uint32(9)) | jnp.uint32(0x3F800000)
            f = jax.lax.bitcast_convert_type(fbits, jnp.float32) \
                - jnp.float32(1.0)
            u = jnp.maximum(f, _TINY)
            w_ref[...] = -jnp.log(u)

            a = jb_ref[...] + ((base + c) * w).astype(jnp.uint32)
            bits_ref[...] = _threefry_bits(a)

        xprev_ref[...] = x_ref[:, (_K - 2) * w:_K * w]

        @pl.when(s == n_steps - 1)
        def _fin():
            bb = best_ref[...]
            col = (btid_ref[...] << shift) + lane
            gmax = jnp.max(bb, axis=1, keepdims=True)
            tok = jnp.min(jnp.where(bb == gmax, col, vocab), axis=1,
                          keepdims=True)
            o_ref[...] = tok

    return body


def kernel(logits):
    b, l, vocab = logits.shape
    rows = b * l
    x = logits.reshape(rows, vocab)
    n_chunks = pl.cdiv(vocab, _W)
    n_steps = pl.cdiv(n_chunks + 2, _K)
    xblocks = pl.cdiv(vocab, _K * _W)

    def x_map(i, s):
        return (i, jnp.minimum(s, xblocks - 1))

    out = pl.pallas_call(
        _make_kernel(vocab, n_steps),
        grid=(rows // _ROWS, n_steps),
        in_specs=[pl.BlockSpec((_ROWS, _K * _W), x_map)],
        out_specs=pl.BlockSpec((_ROWS, 1), lambda i, s: (i, 0)),
        out_shape=jax.ShapeDtypeStruct((rows, 1), jnp.int32),
        scratch_shapes=[
            pltpu.VMEM((_ROWS, _W), jnp.float32),
            pltpu.VMEM((_ROWS, _W), jnp.int32),
            pltpu.VMEM((_ROWS, _W), jnp.uint32),
            pltpu.VMEM((_ROWS, _W), jnp.uint32),
            pltpu.VMEM((_ROWS, _W), jnp.float32),
            pltpu.VMEM((_ROWS, 2 * _W), jnp.float32),
        ],
    )(x)
    return out.reshape(b, l)
